# baseline (speedup 1.0000x reference)
import functools

import jax
import jax.numpy as jnp
from jax import lax
from jax.experimental import pallas as pl
from jax.experimental.pallas import tpu as pltpu
from jax.experimental.pallas import tpu_sc as plsc

N = 10000
E = 320000
D = 128

_NW = 32
_EPW = E // _NW
_C = 80
_NCHUNK = _EPW // _C
_G = _C // 16


def _wid():
    return lax.axis_index("s") * 2 + lax.axis_index("c")


def _enorm_body(src_hbm, dst_hbm, norm_hbm, en_hbm, src_l, dst_l, norm_l, en_l):
    base = _wid() * _EPW
    pltpu.sync_copy(src_hbm.at[pl.ds(base, _EPW)], src_l)
    pltpu.sync_copy(dst_hbm.at[pl.ds(base, _EPW)], dst_l)
    pltpu.sync_copy(norm_hbm, norm_l)

    def body(i, carry):
        off = i * 16
        s16 = src_l[pl.ds(off, 16)]
        d16 = dst_l[pl.ds(off, 16)]
        en = plsc.load_gather(norm_l, [s16]) * plsc.load_gather(norm_l, [d16])
        en_l[pl.ds(off, 16)] = en
        return carry

    lax.fori_loop(0, _EPW // 16, body, 0)
    pltpu.sync_copy(en_l, en_hbm.at[pl.ds(base, _EPW)])


def _enorm(src, dst, norm):
    mesh = plsc.VectorSubcoreMesh(core_axis_name="c", subcore_axis_name="s")
    f = functools.partial(
        pl.kernel,
        out_type=jax.ShapeDtypeStruct((E,), jnp.float32),
        mesh=mesh,
        compiler_params=pltpu.CompilerParams(needs_layout_passes=False),
        scratch_types=[
            pltpu.VMEM((_EPW,), jnp.int32),
            pltpu.VMEM((_EPW,), jnp.int32),
            pltpu.VMEM((N,), jnp.float32),
            pltpu.VMEM((_EPW,), jnp.float32),
        ],
    )(_enorm_body)
    return f(src, dst, norm)


def _edge_body(hf_hbm, ee_hbm, src_hbm, en_hbm, e_hbm,
               src_l, hfr, eeb, enb, sem_g, sem_e, sem_n):
    base = _wid() * _EPW
    pltpu.sync_copy(src_hbm.at[pl.ds(base, _EPW)], src_l)

    def issue(j, b):
        off = j * _C
        pltpu.make_async_copy(
            hf_hbm.at[src_l.at[pl.ds(off, _C)]], hfr.at[b], sem_g.at[b]).start()
        pltpu.make_async_copy(
            ee_hbm.at[pl.ds(base + off, _C)], eeb.at[b], sem_e.at[b]).start()
        pltpu.make_async_copy(
            en_hbm.at[pl.ds(base + off, _C)], enb.at[b], sem_n.at[b]).start()

    def wait(j, b):
        off = j * _C
        pltpu.make_async_copy(
            hf_hbm.at[src_l.at[pl.ds(off, _C)]], hfr.at[b], sem_g.at[b]).wait()
        pltpu.make_async_copy(
            ee_hbm.at[pl.ds(base + off, _C)], eeb.at[b], sem_e.at[b]).wait()
        pltpu.make_async_copy(
            en_hbm.at[pl.ds(base + off, _C)], enb.at[b], sem_n.at[b]).wait()

    issue(0, 0)

    def chunk_body(j, carry):
        b = lax.rem(j, 2)
        bsp = jnp.full((16,), b, jnp.int32)
        wait(j, b)

        @pl.when(j < _NCHUNK - 1)
        def _():
            issue(j + 1, 1 - b)

        def e_body(i2, c2):
            for ii in range(2):
                i = i2 * 2 + ii
                isp = jnp.full((16,), i, jnp.int32)
                env = plsc.load_gather(enb, [bsp, isp])
                for dd in range(8):
                    cols = dd * 16 + lax.iota(jnp.int32, 16)
                    hv = plsc.load_gather(hfr, [bsp, isp, cols])
                    ev = plsc.load_gather(eeb, [bsp, isp, cols])
                    r = jnp.maximum(hv + ev, 0.0) * env
                    plsc.store_scatter(eeb, [bsp, isp, cols], r)
            return c2

        lax.fori_loop(0, _C // 2, e_body, 0)
        pltpu.sync_copy(eeb.at[b], e_hbm.at[pl.ds(base + j * _C, _C)])
        return carry

    lax.fori_loop(0, _NCHUNK, chunk_body, 0)


def _edge_messages(hf, ee, src, en):
    mesh = plsc.VectorSubcoreMesh(core_axis_name="c", subcore_axis_name="s")
    f = functools.partial(
        pl.kernel,
        out_type=jax.ShapeDtypeStruct((E, D), jnp.float32),
        mesh=mesh,
        compiler_params=pltpu.CompilerParams(needs_layout_passes=False),
        scratch_types=[
            pltpu.VMEM((_EPW,), jnp.int32),
            pltpu.VMEM((2, _C, D), jnp.float32),
            pltpu.VMEM((2, _C, D), jnp.float32),
            pltpu.VMEM((2, _C), jnp.float32),
            pltpu.SemaphoreType.DMA((2,)),
            pltpu.SemaphoreType.DMA((2,)),
            pltpu.SemaphoreType.DMA((2,)),
        ],
    )(_edge_body)
    return f(hf, ee, src, en)


def kernel(edge_index, nfeat, efeat, degs, node_emb, lin_W, lin_b, root_emb,
           edge_W, edge_b, bn_gamma, bn_beta, pred_W, pred_b):
    src = edge_index[0]
    dst = edge_index[1]
    norm_n = jnp.power(degs, -0.5)
    en = _enorm(src, dst, norm_n)
    h = node_emb[nfeat]
    L = 3
    for l in range(L):
        hf = h @ lin_W[l] + lin_b[l]
        ee = efeat @ edge_W[l] + edge_b[l]
        e = _edge_messages(hf, ee, src, en)
        ft = jax.ops.segment_sum(e, dst, num_segments=N)
        rst = ft + jax.nn.relu(hf + root_emb[l]) / degs[:, None]
        mean = jnp.mean(rst, axis=0)
        var = jnp.var(rst, axis=0)
        hbn = (rst - mean) / jnp.sqrt(var + 1e-5) * bn_gamma[l] + bn_beta[l]
        if l != L - 1:
            hbn = jax.nn.relu(hbn)
        h = hbn
    hg = jnp.mean(h, axis=0, keepdims=True)
    out = hg @ pred_W + pred_b
    return out


# speedup vs baseline: 1.0235x; 1.0235x over previous
import functools

import jax
import jax.numpy as jnp
from jax import lax
from jax.experimental import pallas as pl
from jax.experimental.pallas import tpu as pltpu
from jax.experimental.pallas import tpu_sc as plsc

N = 10000
E = 320000
D = 128

_NW = 32
_EPW = E // _NW
_C = 80
_NCHUNK = _EPW // _C
_G = _C // 16


def _wid():
    return lax.axis_index("s") * 2 + lax.axis_index("c")


def _enorm_body(src_hbm, dst_hbm, norm_hbm, en_hbm, src_l, dst_l, norm_l, en_l):
    base = _wid() * _EPW
    pltpu.sync_copy(src_hbm.at[pl.ds(base, _EPW)], src_l)
    pltpu.sync_copy(dst_hbm.at[pl.ds(base, _EPW)], dst_l)
    pltpu.sync_copy(norm_hbm, norm_l)

    def body(i, carry):
        off = i * 16
        s16 = src_l[pl.ds(off, 16)]
        d16 = dst_l[pl.ds(off, 16)]
        en = plsc.load_gather(norm_l, [s16]) * plsc.load_gather(norm_l, [d16])
        en_l[pl.ds(off, 16)] = en
        return carry

    lax.fori_loop(0, _EPW // 16, body, 0)
    pltpu.sync_copy(en_l, en_hbm.at[pl.ds(base, _EPW)])


def _enorm(src, dst, norm):
    mesh = plsc.VectorSubcoreMesh(core_axis_name="c", subcore_axis_name="s")
    f = functools.partial(
        pl.kernel,
        out_type=jax.ShapeDtypeStruct((E,), jnp.float32),
        mesh=mesh,
        compiler_params=pltpu.CompilerParams(needs_layout_passes=False),
        scratch_types=[
            pltpu.VMEM((_EPW,), jnp.int32),
            pltpu.VMEM((_EPW,), jnp.int32),
            pltpu.VMEM((N,), jnp.float32),
            pltpu.VMEM((_EPW,), jnp.float32),
        ],
    )(_enorm_body)
    return f(src, dst, norm)


def _edge_body(hf_hbm, ee_hbm, src_hbm, en_hbm, e_hbm,
               src_l, hfr, eeb, enb, sem_g, sem_e, sem_n):
    base = _wid() * _EPW
    pltpu.sync_copy(src_hbm.at[pl.ds(base, _EPW)], src_l)

    def issue(j, b):
        off = j * _C
        pltpu.make_async_copy(
            hf_hbm.at[src_l.at[pl.ds(off, _C)]], hfr.at[b], sem_g.at[b]).start()
        pltpu.make_async_copy(
            ee_hbm.at[pl.ds(base + off, _C)], eeb.at[b], sem_e.at[b]).start()
        pltpu.make_async_copy(
            en_hbm.at[pl.ds(base + off, _C)], enb.at[b], sem_n.at[b]).start()

    def wait(j, b):
        off = j * _C
        pltpu.make_async_copy(
            hf_hbm.at[src_l.at[pl.ds(off, _C)]], hfr.at[b], sem_g.at[b]).wait()
        pltpu.make_async_copy(
            ee_hbm.at[pl.ds(base + off, _C)], eeb.at[b], sem_e.at[b]).wait()
        pltpu.make_async_copy(
            en_hbm.at[pl.ds(base + off, _C)], enb.at[b], sem_n.at[b]).wait()

    issue(0, 0)

    def chunk_body(j, carry):
        b = lax.rem(j, 2)
        bsp = jnp.full((16,), b, jnp.int32)
        wait(j, b)

        @pl.when(j < _NCHUNK - 1)
        def _():
            issue(j + 1, 1 - b)

        def e_body(i2, c2):
            for ii in range(4):
                i = i2 * 4 + ii
                isp = jnp.full((16,), i, jnp.int32)
                env = plsc.load_gather(enb, [bsp, isp])
                for dd in range(8):
                    cols = dd * 16 + lax.iota(jnp.int32, 16)
                    hv = plsc.load_gather(hfr, [bsp, isp, cols])
                    ev = plsc.load_gather(eeb, [bsp, isp, cols])
                    r = jnp.maximum(hv + ev, 0.0) * env
                    plsc.store_scatter(eeb, [bsp, isp, cols], r)
            return c2

        lax.fori_loop(0, _C // 4, e_body, 0)
        pltpu.sync_copy(eeb.at[b], e_hbm.at[pl.ds(base + j * _C, _C)])
        return carry

    lax.fori_loop(0, _NCHUNK, chunk_body, 0)


def _edge_messages(hf, ee, src, en):
    mesh = plsc.VectorSubcoreMesh(core_axis_name="c", subcore_axis_name="s")
    f = functools.partial(
        pl.kernel,
        out_type=jax.ShapeDtypeStruct((E, D), jnp.float32),
        mesh=mesh,
        compiler_params=pltpu.CompilerParams(needs_layout_passes=False),
        scratch_types=[
            pltpu.VMEM((_EPW,), jnp.int32),
            pltpu.VMEM((2, _C, D), jnp.float32),
            pltpu.VMEM((2, _C, D), jnp.float32),
            pltpu.VMEM((2, _C), jnp.float32),
            pltpu.SemaphoreType.DMA((2,)),
            pltpu.SemaphoreType.DMA((2,)),
            pltpu.SemaphoreType.DMA((2,)),
        ],
    )(_edge_body)
    return f(hf, ee, src, en)


def kernel(edge_index, nfeat, efeat, degs, node_emb, lin_W, lin_b, root_emb,
           edge_W, edge_b, bn_gamma, bn_beta, pred_W, pred_b):
    src = edge_index[0]
    dst = edge_index[1]
    norm_n = jnp.power(degs, -0.5)
    en = _enorm(src, dst, norm_n)
    h = node_emb[nfeat]
    L = 3
    for l in range(L):
        hf = h @ lin_W[l] + lin_b[l]
        ee = efeat @ edge_W[l] + edge_b[l]
        e = _edge_messages(hf, ee, src, en)
        ft = jax.ops.segment_sum(e, dst, num_segments=N)
        rst = ft + jax.nn.relu(hf + root_emb[l]) / degs[:, None]
        mean = jnp.mean(rst, axis=0)
        var = jnp.var(rst, axis=0)
        hbn = (rst - mean) / jnp.sqrt(var + 1e-5) * bn_gamma[l] + bn_beta[l]
        if l != L - 1:
            hbn = jax.nn.relu(hbn)
        h = hbn
    hg = jnp.mean(h, axis=0, keepdims=True)
    out = hg @ pred_W + pred_b
    return out


# speedup vs baseline: 1.0622x; 1.0379x over previous
import functools

import jax
import jax.numpy as jnp
from jax import lax
from jax.experimental import pallas as pl
from jax.experimental.pallas import tpu as pltpu
from jax.experimental.pallas import tpu_sc as plsc

N = 10000
E = 320000
D = 128

_NW = 32
_EPW = E // _NW
_C = 80
_NCHUNK = _EPW // _C
_G = _C // 16


def _wid():
    return lax.axis_index("s") * 2 + lax.axis_index("c")


def _enorm_body(src_hbm, dst_hbm, norm_hbm, en_hbm, src_l, dst_l, norm_l, en_l):
    base = _wid() * _EPW
    pltpu.sync_copy(src_hbm.at[pl.ds(base, _EPW)], src_l)
    pltpu.sync_copy(dst_hbm.at[pl.ds(base, _EPW)], dst_l)
    pltpu.sync_copy(norm_hbm, norm_l)

    def body(i, carry):
        off = i * 16
        s16 = src_l[pl.ds(off, 16)]
        d16 = dst_l[pl.ds(off, 16)]
        en = plsc.load_gather(norm_l, [s16]) * plsc.load_gather(norm_l, [d16])
        en_l[pl.ds(off, 16)] = en
        return carry

    lax.fori_loop(0, _EPW // 16, body, 0)
    pltpu.sync_copy(en_l, en_hbm.at[pl.ds(base, _EPW)])


def _enorm(src, dst, norm):
    mesh = plsc.VectorSubcoreMesh(core_axis_name="c", subcore_axis_name="s")
    f = functools.partial(
        pl.kernel,
        out_type=jax.ShapeDtypeStruct((E,), jnp.float32),
        mesh=mesh,
        compiler_params=pltpu.CompilerParams(needs_layout_passes=False),
        scratch_types=[
            pltpu.VMEM((_EPW,), jnp.int32),
            pltpu.VMEM((_EPW,), jnp.int32),
            pltpu.VMEM((N,), jnp.float32),
            pltpu.VMEM((_EPW,), jnp.float32),
        ],
    )(_enorm_body)
    return f(src, dst, norm)


def _edge_body(hf_hbm, ee_hbm, src_hbm, en_hbm, e_hbm, src_l,
               hfrA, hfrB, eebA, eebB, enA, enB,
               sgA, sgB, seA, seB, snA, snB):
    base = _wid() * _EPW
    pltpu.sync_copy(src_hbm.at[pl.ds(base, _EPW)], src_l)
    bufs = ((hfrA, eebA, enA, sgA, seA, snA),
            (hfrB, eebB, enB, sgB, seB, snB))

    def issue(j, b):
        hfr, eeb, enb, sg, se, sn = bufs[b]
        off = j * _C
        pltpu.make_async_copy(
            hf_hbm.at[src_l.at[pl.ds(off, _C)]], hfr, sg).start()
        pltpu.make_async_copy(
            ee_hbm.at[pl.ds(base + off, _C)], eeb, se).start()
        pltpu.make_async_copy(
            en_hbm.at[pl.ds(base + off, _C)], enb, sn).start()

    def wait(j, b):
        hfr, eeb, enb, sg, se, sn = bufs[b]
        off = j * _C
        pltpu.make_async_copy(
            hf_hbm.at[src_l.at[pl.ds(off, _C)]], hfr, sg).wait()
        pltpu.make_async_copy(
            ee_hbm.at[pl.ds(base + off, _C)], eeb, se).wait()
        pltpu.make_async_copy(
            en_hbm.at[pl.ds(base + off, _C)], enb, sn).wait()

    def process(j, b):
        hfr, eeb, enb, *_ = bufs[b]
        wait(j, b)

        @pl.when(j + 1 < _NCHUNK)
        def _():
            issue(j + 1, 1 - b)

        def e_body(i2, c2):
            for ii in range(4):
                i = i2 * 4 + ii
                env = plsc.load_gather(enb, [jnp.full((16,), i, jnp.int32)])
                for dd in range(8):
                    sl = pl.ds(dd * 16, 16)
                    r = jnp.maximum(hfr[i, sl] + eeb[i, sl], 0.0) * env
                    eeb[i, sl] = r
            return c2

        lax.fori_loop(0, _C // 4, e_body, 0)
        pltpu.sync_copy(eeb, e_hbm.at[pl.ds(base + j * _C, _C)])

    issue(0, 0)

    def chunk2(j2, carry):
        process(2 * j2, 0)
        process(2 * j2 + 1, 1)
        return carry

    lax.fori_loop(0, _NCHUNK // 2, chunk2, 0)
    process(_NCHUNK - 1, 0)


def _edge_messages(hf, ee, src, en):
    mesh = plsc.VectorSubcoreMesh(core_axis_name="c", subcore_axis_name="s")
    f = functools.partial(
        pl.kernel,
        out_type=jax.ShapeDtypeStruct((E, D), jnp.float32),
        mesh=mesh,
        compiler_params=pltpu.CompilerParams(needs_layout_passes=False),
        scratch_types=[
            pltpu.VMEM((_EPW,), jnp.int32),
            pltpu.VMEM((_C, D), jnp.float32),
            pltpu.VMEM((_C, D), jnp.float32),
            pltpu.VMEM((_C, D), jnp.float32),
            pltpu.VMEM((_C, D), jnp.float32),
            pltpu.VMEM((_C,), jnp.float32),
            pltpu.VMEM((_C,), jnp.float32),
            pltpu.SemaphoreType.DMA,
            pltpu.SemaphoreType.DMA,
            pltpu.SemaphoreType.DMA,
            pltpu.SemaphoreType.DMA,
            pltpu.SemaphoreType.DMA,
            pltpu.SemaphoreType.DMA,
        ],
    )(_edge_body)
    return f(hf, ee, src, en)


def kernel(edge_index, nfeat, efeat, degs, node_emb, lin_W, lin_b, root_emb,
           edge_W, edge_b, bn_gamma, bn_beta, pred_W, pred_b):
    src = edge_index[0]
    dst = edge_index[1]
    norm_n = jnp.power(degs, -0.5)
    en = _enorm(src, dst, norm_n)
    h = node_emb[nfeat]
    L = 3
    for l in range(L):
        hf = h @ lin_W[l] + lin_b[l]
        ee = efeat @ edge_W[l] + edge_b[l]
        e = _edge_messages(hf, ee, src, en)
        ft = jax.ops.segment_sum(e, dst, num_segments=N)
        rst = ft + jax.nn.relu(hf + root_emb[l]) / degs[:, None]
        mean = jnp.mean(rst, axis=0)
        var = jnp.var(rst, axis=0)
        hbn = (rst - mean) / jnp.sqrt(var + 1e-5) * bn_gamma[l] + bn_beta[l]
        if l != L - 1:
            hbn = jax.nn.relu(hbn)
        h = hbn
    hg = jnp.mean(h, axis=0, keepdims=True)
    out = hg @ pred_W + pred_b
    return out


# speedup vs baseline: 1.4015x; 1.3194x over previous
import functools

import jax
import jax.numpy as jnp
from jax import lax
from jax.experimental import pallas as pl
from jax.experimental.pallas import tpu as pltpu
from jax.experimental.pallas import tpu_sc as plsc

N = 10000
E = 320000
D = 128

_NW = 32
_EPW = E // _NW
_C = 80
_NCHUNK = _EPW // _C
_G = _C // 16


def _wid():
    return lax.axis_index("s") * 2 + lax.axis_index("c")


def _enorm_body(src_hbm, dst_hbm, norm_hbm, en_hbm, src_l, dst_l, norm_l, en_l):
    base = _wid() * _EPW
    pltpu.sync_copy(src_hbm.at[pl.ds(base, _EPW)], src_l)
    pltpu.sync_copy(dst_hbm.at[pl.ds(base, _EPW)], dst_l)
    pltpu.sync_copy(norm_hbm, norm_l)

    def body(i, carry):
        off = i * 16
        s16 = src_l[pl.ds(off, 16)]
        d16 = dst_l[pl.ds(off, 16)]
        en = plsc.load_gather(norm_l, [s16]) * plsc.load_gather(norm_l, [d16])
        en_l[pl.ds(off, 16)] = en
        return carry

    lax.fori_loop(0, _EPW // 16, body, 0)
    pltpu.sync_copy(en_l, en_hbm.at[pl.ds(base, _EPW)])


def _enorm(src, dst, norm):
    mesh = plsc.VectorSubcoreMesh(core_axis_name="c", subcore_axis_name="s")
    f = functools.partial(
        pl.kernel,
        out_type=jax.ShapeDtypeStruct((E,), jnp.float32),
        mesh=mesh,
        compiler_params=pltpu.CompilerParams(needs_layout_passes=False),
        scratch_types=[
            pltpu.VMEM((_EPW,), jnp.int32),
            pltpu.VMEM((_EPW,), jnp.int32),
            pltpu.VMEM((N,), jnp.float32),
            pltpu.VMEM((_EPW,), jnp.float32),
        ],
    )(_enorm_body)
    return f(src, dst, norm)


def _edge_body(hf_hbm, ee_hbm, src_hbm, en_hbm, e_hbm, src_l, en_l,
               hfrA, hfrB, eebA, eebB, eobA, eobB,
               sgA, sgB, seA, seB, soA, soB):
    base = _wid() * _EPW
    pltpu.sync_copy(src_hbm.at[pl.ds(base, _EPW)], src_l)
    pltpu.sync_copy(en_hbm.at[pl.ds(base, _EPW)], en_l)
    bufs = ((hfrA, eebA, eobA, sgA, seA, soA),
            (hfrB, eebB, eobB, sgB, seB, soB))

    def issue(j, b):
        hfr, eeb, eob, sg, se, so = bufs[b]
        off = j * _C
        pltpu.make_async_copy(
            hf_hbm.at[src_l.at[pl.ds(off, _C)]], hfr, sg).start()
        pltpu.make_async_copy(
            ee_hbm.at[pl.ds(base + off, _C)], eeb, se).start()

    def wait(j, b):
        hfr, eeb, eob, sg, se, so = bufs[b]
        off = j * _C
        pltpu.make_async_copy(
            hf_hbm.at[src_l.at[pl.ds(off, _C)]], hfr, sg).wait()
        pltpu.make_async_copy(
            ee_hbm.at[pl.ds(base + off, _C)], eeb, se).wait()

    def wait_out(j, b):
        hfr, eeb, eob, sg, se, so = bufs[b]
        pltpu.make_async_copy(
            eob, e_hbm.at[pl.ds(base + j * _C, _C)], so).wait()

    def process(j, b):
        hfr, eeb, eob, sg, se, so = bufs[b]
        wait(j, b)

        @pl.when(j + 1 < _NCHUNK)
        def _():
            issue(j + 1, 1 - b)

        @pl.when(j >= 2)
        def _():
            wait_out(j - 2, b)

        def e_body(o, c2):
            en16 = en_l[pl.ds(j * _C + o * 16, 16)]
            for ii in range(16):
                i = o * 16 + ii
                env = jnp.full((16,), en16[ii], jnp.float32)
                for dd in range(8):
                    sl = pl.ds(dd * 16, 16)
                    r = jnp.maximum(hfr[i, sl] + eeb[i, sl], 0.0) * env
                    eob[i, sl] = r
            return c2

        lax.fori_loop(0, _G, e_body, 0)
        pltpu.make_async_copy(
            eob, e_hbm.at[pl.ds(base + j * _C, _C)], so).start()

    issue(0, 0)

    def chunk2(j2, carry):
        process(2 * j2, 0)
        process(2 * j2 + 1, 1)
        return carry

    lax.fori_loop(0, _NCHUNK // 2, chunk2, 0)
    process(_NCHUNK - 1, 0)
    wait_out(_NCHUNK - 2, 1)
    wait_out(_NCHUNK - 1, 0)


def _edge_messages(hf, ee, src, en):
    mesh = plsc.VectorSubcoreMesh(core_axis_name="c", subcore_axis_name="s")
    f = functools.partial(
        pl.kernel,
        out_type=jax.ShapeDtypeStruct((E, D), jnp.float32),
        mesh=mesh,
        compiler_params=pltpu.CompilerParams(needs_layout_passes=False),
        scratch_types=[
            pltpu.VMEM((_EPW,), jnp.int32),
            pltpu.VMEM((_EPW,), jnp.float32),
            pltpu.VMEM((_C, D), jnp.float32),
            pltpu.VMEM((_C, D), jnp.float32),
            pltpu.VMEM((_C, D), jnp.float32),
            pltpu.VMEM((_C, D), jnp.float32),
            pltpu.VMEM((_C, D), jnp.float32),
            pltpu.VMEM((_C, D), jnp.float32),
            pltpu.SemaphoreType.DMA,
            pltpu.SemaphoreType.DMA,
            pltpu.SemaphoreType.DMA,
            pltpu.SemaphoreType.DMA,
            pltpu.SemaphoreType.DMA,
            pltpu.SemaphoreType.DMA,
        ],
    )(_edge_body)
    return f(hf, ee, src, en)


def kernel(edge_index, nfeat, efeat, degs, node_emb, lin_W, lin_b, root_emb,
           edge_W, edge_b, bn_gamma, bn_beta, pred_W, pred_b):
    src = edge_index[0]
    dst = edge_index[1]
    norm_n = jnp.power(degs, -0.5)
    en = _enorm(src, dst, norm_n)
    h = node_emb[nfeat]
    L = 3
    for l in range(L):
        hf = h @ lin_W[l] + lin_b[l]
        ee = efeat @ edge_W[l] + edge_b[l]
        e = _edge_messages(hf, ee, src, en)
        ft = jax.ops.segment_sum(e, dst, num_segments=N)
        rst = ft + jax.nn.relu(hf + root_emb[l]) / degs[:, None]
        mean = jnp.mean(rst, axis=0)
        var = jnp.var(rst, axis=0)
        hbn = (rst - mean) / jnp.sqrt(var + 1e-5) * bn_gamma[l] + bn_beta[l]
        if l != L - 1:
            hbn = jax.nn.relu(hbn)
        h = hbn
    hg = jnp.mean(h, axis=0, keepdims=True)
    out = hg @ pred_W + pred_b
    return out


# speedup vs baseline: 1.4207x; 1.0137x over previous
import functools

import jax
import jax.numpy as jnp
from jax import lax
from jax.experimental import pallas as pl
from jax.experimental.pallas import tpu as pltpu
from jax.experimental.pallas import tpu_sc as plsc

N = 10000
E = 320000
D = 128

_NW = 32
_EPW = E // _NW
_C = 80
_NCHUNK = _EPW // _C
_G = _C // 16


def _wid():
    return lax.axis_index("s") * 2 + lax.axis_index("c")


def _enorm_body(src_hbm, dst_hbm, norm_hbm, en_hbm, src_l, dst_l, norm_l, en_l):
    base = _wid() * _EPW
    pltpu.sync_copy(src_hbm.at[pl.ds(base, _EPW)], src_l)
    pltpu.sync_copy(dst_hbm.at[pl.ds(base, _EPW)], dst_l)
    pltpu.sync_copy(norm_hbm, norm_l)

    def body(i, carry):
        off = i * 16
        s16 = src_l[pl.ds(off, 16)]
        d16 = dst_l[pl.ds(off, 16)]
        en = plsc.load_gather(norm_l, [s16]) * plsc.load_gather(norm_l, [d16])
        en_l[pl.ds(off, 16)] = en
        return carry

    lax.fori_loop(0, _EPW // 16, body, 0)
    pltpu.sync_copy(en_l, en_hbm.at[pl.ds(base, _EPW)])


def _enorm(src, dst, norm):
    mesh = plsc.VectorSubcoreMesh(core_axis_name="c", subcore_axis_name="s")
    f = functools.partial(
        pl.kernel,
        out_type=jax.ShapeDtypeStruct((E,), jnp.float32),
        mesh=mesh,
        compiler_params=pltpu.CompilerParams(needs_layout_passes=False),
        scratch_types=[
            pltpu.VMEM((_EPW,), jnp.int32),
            pltpu.VMEM((_EPW,), jnp.int32),
            pltpu.VMEM((N,), jnp.float32),
            pltpu.VMEM((_EPW,), jnp.float32),
        ],
    )(_enorm_body)
    return f(src, dst, norm)


def _edge_body(hf_hbm, ee_hbm, src_hbm, en_hbm, e_hbm, src_l, en_l,
               hfrA, hfrB, hfrC, eebA, eebB, eebC, eobA, eobB, eobC,
               sgA, sgB, sgC, seA, seB, seC, soA, soB, soC):
    base = _wid() * _EPW
    pltpu.sync_copy(src_hbm.at[pl.ds(base, _EPW)], src_l)
    pltpu.sync_copy(en_hbm.at[pl.ds(base, _EPW)], en_l)
    bufs = ((hfrA, eebA, eobA, sgA, seA, soA),
            (hfrB, eebB, eobB, sgB, seB, soB),
            (hfrC, eebC, eobC, sgC, seC, soC))

    def issue(j, b):
        hfr, eeb, eob, sg, se, so = bufs[b]
        off = j * _C
        pltpu.make_async_copy(
            hf_hbm.at[src_l.at[pl.ds(off, _C)]], hfr, sg).start()
        pltpu.make_async_copy(
            ee_hbm.at[pl.ds(base + off, _C)], eeb, se).start()

    def wait(j, b):
        hfr, eeb, eob, sg, se, so = bufs[b]
        off = j * _C
        pltpu.make_async_copy(
            hf_hbm.at[src_l.at[pl.ds(off, _C)]], hfr, sg).wait()
        pltpu.make_async_copy(
            ee_hbm.at[pl.ds(base + off, _C)], eeb, se).wait()

    def wait_out(j, b):
        hfr, eeb, eob, sg, se, so = bufs[b]
        pltpu.make_async_copy(
            eob, e_hbm.at[pl.ds(base + j * _C, _C)], so).wait()

    def process(j, b):
        hfr, eeb, eob, sg, se, so = bufs[b]
        wait(j, b)

        @pl.when(j + 2 < _NCHUNK)
        def _():
            issue(j + 2, (b + 2) % 3)

        @pl.when(j >= 3)
        def _():
            wait_out(j - 3, b)

        def e_body(o, c2):
            en16 = en_l[pl.ds(j * _C + o * 16, 16)]
            for ii in range(16):
                i = o * 16 + ii
                env = jnp.full((16,), en16[ii], jnp.float32)
                for dd in range(8):
                    sl = pl.ds(dd * 16, 16)
                    r = jnp.maximum(hfr[i, sl] + eeb[i, sl], 0.0) * env
                    eob[i, sl] = r
            return c2

        lax.fori_loop(0, _G, e_body, 0)
        pltpu.make_async_copy(
            eob, e_hbm.at[pl.ds(base + j * _C, _C)], so).start()

    issue(0, 0)
    issue(1, 1)

    def chunk3(j3, carry):
        process(3 * j3, 0)
        process(3 * j3 + 1, 1)
        process(3 * j3 + 2, 2)
        return carry

    lax.fori_loop(0, _NCHUNK // 3, chunk3, 0)
    process(_NCHUNK - 2, 0)
    process(_NCHUNK - 1, 1)
    wait_out(_NCHUNK - 3, 2)
    wait_out(_NCHUNK - 2, 0)
    wait_out(_NCHUNK - 1, 1)


def _edge_messages(hf, ee, src, en):
    mesh = plsc.VectorSubcoreMesh(core_axis_name="c", subcore_axis_name="s")
    f = functools.partial(
        pl.kernel,
        out_type=jax.ShapeDtypeStruct((E, D), jnp.float32),
        mesh=mesh,
        compiler_params=pltpu.CompilerParams(needs_layout_passes=False),
        scratch_types=[
            pltpu.VMEM((_EPW,), jnp.int32),
            pltpu.VMEM((_EPW,), jnp.float32),
            pltpu.VMEM((_C, D), jnp.float32),
            pltpu.VMEM((_C, D), jnp.float32),
            pltpu.VMEM((_C, D), jnp.float32),
            pltpu.VMEM((_C, D), jnp.float32),
            pltpu.VMEM((_C, D), jnp.float32),
            pltpu.VMEM((_C, D), jnp.float32),
            pltpu.VMEM((_C, D), jnp.float32),
            pltpu.VMEM((_C, D), jnp.float32),
            pltpu.VMEM((_C, D), jnp.float32),
            pltpu.SemaphoreType.DMA,
            pltpu.SemaphoreType.DMA,
            pltpu.SemaphoreType.DMA,
            pltpu.SemaphoreType.DMA,
            pltpu.SemaphoreType.DMA,
            pltpu.SemaphoreType.DMA,
            pltpu.SemaphoreType.DMA,
            pltpu.SemaphoreType.DMA,
            pltpu.SemaphoreType.DMA,
        ],
    )(_edge_body)
    return f(hf, ee, src, en)


def kernel(edge_index, nfeat, efeat, degs, node_emb, lin_W, lin_b, root_emb,
           edge_W, edge_b, bn_gamma, bn_beta, pred_W, pred_b):
    src = edge_index[0]
    dst = edge_index[1]
    norm_n = jnp.power(degs, -0.5)
    en = _enorm(src, dst, norm_n)
    h = node_emb[nfeat]
    L = 3
    for l in range(L):
        hf = h @ lin_W[l] + lin_b[l]
        ee = efeat @ edge_W[l] + edge_b[l]
        e = _edge_messages(hf, ee, src, en)
        ft = jax.ops.segment_sum(e, dst, num_segments=N)
        rst = ft + jax.nn.relu(hf + root_emb[l]) / degs[:, None]
        mean = jnp.mean(rst, axis=0)
        var = jnp.var(rst, axis=0)
        hbn = (rst - mean) / jnp.sqrt(var + 1e-5) * bn_gamma[l] + bn_beta[l]
        if l != L - 1:
            hbn = jax.nn.relu(hbn)
        h = hbn
    hg = jnp.mean(h, axis=0, keepdims=True)
    out = hg @ pred_W + pred_b
    return out
